# baseline (speedup 1.0000x reference)
import functools

import jax
import jax.numpy as jnp
from jax import lax
from jax.experimental import pallas as pl
from jax.experimental.pallas import tpu as pltpu
from jax.experimental.pallas import tpu_sc as plsc

_B = 16384
_C = 8
_NC = 1
_NS = 16
_NW = _NC * _NS
_BPW = _B // _NW
_L = 16
_STEPS = _BPW // _L


def _sc_body(data_hbm, maskt_hbm, out_hbm, data_v, mask_v, out_v, dsem, msem):
    wid = lax.axis_index("s") * _NC + lax.axis_index("c")
    base = wid * _BPW
    dcp = pltpu.async_copy(data_hbm.at[pl.ds(base, _BPW)], data_v, dsem)
    mcp = pltpu.async_copy(maskt_hbm.at[:, pl.ds(base, _BPW)], mask_v, msem)
    dcp.wait()
    mcp.wait()

    rows = lax.iota(jnp.int32, _L)

    @pl.loop(0, _STEPS, unroll=4)
    def _gather(j):
        cols = data_v[pl.ds(j * _L, _L)] - 1
        out_v[pl.ds(j * _L, _L)] = plsc.load_gather(
            mask_v, [cols, rows + j * _L]
        )

    pltpu.sync_copy(out_v, out_hbm.at[pl.ds(base, _BPW)])


_sc_call = functools.partial(
    pl.kernel,
    out_type=jax.ShapeDtypeStruct((_B,), jnp.float32),
    mesh=plsc.VectorSubcoreMesh(
        core_axis_name="c", subcore_axis_name="s", num_cores=_NC
    ),
    compiler_params=pltpu.CompilerParams(needs_layout_passes=False),
    scratch_types=[
        pltpu.VMEM((_BPW,), jnp.int32),
        pltpu.VMEM((_C, _BPW), jnp.float32),
        pltpu.VMEM((_BPW,), jnp.float32),
        pltpu.SemaphoreType.DMA,
        pltpu.SemaphoreType.DMA,
    ],
)(_sc_body)


def kernel(data, color_mask):
    return _sc_call(data, color_mask.T)


# speedup vs baseline: 1.0130x; 1.0130x over previous
import functools

import jax
import jax.numpy as jnp
from jax import lax
from jax.experimental import pallas as pl
from jax.experimental.pallas import tpu as pltpu
from jax.experimental.pallas import tpu_sc as plsc

_B = 16384
_C = 8
_NC = 1
_NS = 16
_NW = _NC * _NS
_BPW = _B // _NW
_L = 16
_STEPS = _BPW // _L


def _sc_body(data_hbm, maskt_hbm, out_hbm, data_v, mask_v, out_v, dsem, msem):
    wid = lax.axis_index("s") * _NC + lax.axis_index("c")
    base = wid * _BPW
    dcp = pltpu.async_copy(data_hbm.at[pl.ds(base, _BPW)], data_v, dsem)
    mcp = pltpu.async_copy(maskt_hbm.at[:, pl.ds(base, _BPW)], mask_v, msem)
    dcp.wait()
    mcp.wait()

    rows = lax.iota(jnp.int32, _L)

    @pl.loop(0, _STEPS, unroll=2)
    def _gather(j):
        cols = data_v[pl.ds(j * _L, _L)] - 1
        out_v[pl.ds(j * _L, _L)] = plsc.load_gather(
            mask_v, [cols, rows + j * _L]
        )

    pltpu.sync_copy(out_v, out_hbm.at[pl.ds(base, _BPW)])


_sc_call = functools.partial(
    pl.kernel,
    out_type=jax.ShapeDtypeStruct((_B,), jnp.float32),
    mesh=plsc.VectorSubcoreMesh(
        core_axis_name="c", subcore_axis_name="s", num_cores=_NC
    ),
    compiler_params=pltpu.CompilerParams(needs_layout_passes=False),
    scratch_types=[
        pltpu.VMEM((_BPW,), jnp.int32),
        pltpu.VMEM((_C, _BPW), jnp.float32),
        pltpu.VMEM((_BPW,), jnp.float32),
        pltpu.SemaphoreType.DMA,
        pltpu.SemaphoreType.DMA,
    ],
)(_sc_body)


def kernel(data, color_mask):
    return _sc_call(data, color_mask.T)


# speedup vs baseline: 1.0154x; 1.0024x over previous
import functools

import jax
import jax.numpy as jnp
from jax import lax
from jax.experimental import pallas as pl
from jax.experimental.pallas import tpu as pltpu
from jax.experimental.pallas import tpu_sc as plsc

_B = 16384
_C = 8
_NC = 1
_NS = 16
_NW = _NC * _NS
_BPW = _B // _NW
_L = 16
_STEPS = _BPW // _L


def _sc_body(data_hbm, maskt_hbm, out_hbm, data_v, mask_v, out_v, dsem, msem):
    wid = lax.axis_index("s") * _NC + lax.axis_index("c")
    base = wid * _BPW
    dcp = pltpu.async_copy(data_hbm.at[pl.ds(base, _BPW)], data_v, dsem)
    mcp = pltpu.async_copy(maskt_hbm.at[:, pl.ds(base, _BPW)], mask_v, msem)
    dcp.wait()
    mcp.wait()

    rows = lax.iota(jnp.int32, _L)

    @pl.loop(0, _STEPS, unroll=1)
    def _gather(j):
        cols = data_v[pl.ds(j * _L, _L)] - 1
        out_v[pl.ds(j * _L, _L)] = plsc.load_gather(
            mask_v, [cols, rows + j * _L]
        )

    pltpu.sync_copy(out_v, out_hbm.at[pl.ds(base, _BPW)])


_sc_call = functools.partial(
    pl.kernel,
    out_type=jax.ShapeDtypeStruct((_B,), jnp.float32),
    mesh=plsc.VectorSubcoreMesh(
        core_axis_name="c", subcore_axis_name="s", num_cores=_NC
    ),
    compiler_params=pltpu.CompilerParams(needs_layout_passes=False),
    scratch_types=[
        pltpu.VMEM((_BPW,), jnp.int32),
        pltpu.VMEM((_C, _BPW), jnp.float32),
        pltpu.VMEM((_BPW,), jnp.float32),
        pltpu.SemaphoreType.DMA,
        pltpu.SemaphoreType.DMA,
    ],
)(_sc_body)


def kernel(data, color_mask):
    return _sc_call(data, color_mask.T)


# speedup vs baseline: 1.0171x; 1.0016x over previous
import functools

import jax
import jax.numpy as jnp
from jax import lax
from jax.experimental import pallas as pl
from jax.experimental.pallas import tpu as pltpu
from jax.experimental.pallas import tpu_sc as plsc

_B = 16384
_C = 8
_NC = 1
_NS = 16
_NW = _NC * _NS
_BPW = _B // _NW
_L = 16
_STEPS = _BPW // _L


def _sc_body(data_hbm, maskt_hbm, out_hbm, data_v, mask_v, out_v, dsem, msem):
    wid = lax.axis_index("s") * _NC + lax.axis_index("c")
    base = wid * _BPW
    dcp = pltpu.async_copy(data_hbm.at[pl.ds(base, _BPW)], data_v, dsem)
    mcp = pltpu.async_copy(maskt_hbm.at[:, pl.ds(base, _BPW)], mask_v, msem)
    dcp.wait()
    mcp.wait()

    rows = lax.iota(jnp.int32, _L)

    @pl.loop(0, _STEPS, unroll=2)
    def _gather(j):
        cols = data_v[pl.ds(j * _L, _L)] - 1
        out_v[pl.ds(j * _L, _L)] = plsc.load_gather(
            mask_v, [cols, rows + j * _L]
        )

    pltpu.sync_copy(out_v, out_hbm.at[pl.ds(base, _BPW)])


_sc_call = functools.partial(
    pl.kernel,
    out_type=jax.ShapeDtypeStruct((_B,), jnp.float32),
    mesh=plsc.VectorSubcoreMesh(
        core_axis_name="c", subcore_axis_name="s", num_cores=_NC
    ),
    compiler_params=pltpu.CompilerParams(needs_layout_passes=False),
    scratch_types=[
        pltpu.VMEM((_BPW,), jnp.int32),
        pltpu.VMEM((_C, _BPW), jnp.float32),
        pltpu.VMEM((_BPW,), jnp.float32),
        pltpu.SemaphoreType.DMA,
        pltpu.SemaphoreType.DMA,
    ],
)(_sc_body)


def kernel(data, color_mask):
    return _sc_call(data, color_mask.T)
